# Initial kernel scaffold; baseline (speedup 1.0000x reference)
import jax
import jax.numpy as jnp
from jax.experimental import pallas as pl


def kernel(pts, params):
    raise NotImplementedError("write your pallas kernel here")


# speedup vs baseline: 4.8713x; 4.8713x over previous
import jax
import jax.numpy as jnp
from jax import lax
from jax.experimental import pallas as pl

_BN_EPS = 1e-5
_INTERPRET = False


def _bn_relu(x, stats, g, b, count):
    inv = 1.0 / count
    mean = stats[0:1, :] * inv
    var = stats[1:2, :] * inv - mean * mean
    return jnp.maximum(
        g * (x - mean) / jnp.sqrt(var + _BN_EPS) + b, 0.0)


def _mlp(xs, Ws, bias, act, emit_stats, mblk):
    M = xs[0].shape[0]
    Cout = Ws[0].shape[1]
    nx = len(xs)
    has_act = act is not None
    assert not (has_act and nx != 1)
    grid = (M // mblk,)

    def body(*refs):
        it = iter(refs)
        x_refs = [next(it) for _ in range(nx)]
        w_refs = [next(it) for _ in range(nx)]
        bias_ref = next(it)
        if has_act:
            st_ref = next(it)
            g_ref = next(it)
            b_ref = next(it)
        out_ref = next(it)
        if emit_stats:
            so_ref = next(it)
        acc = None
        for t in range(nx):
            xv = x_refs[t][...]
            if has_act:
                xv = _bn_relu(xv, st_ref[...], g_ref[...], b_ref[...], act[3])
            p = jnp.dot(xv, w_refs[t][...], preferred_element_type=jnp.float32)
            acc = p if acc is None else acc + p
        y = acc + bias_ref[...]
        out_ref[...] = y
        if emit_stats:
            s0 = jnp.sum(y, axis=0, keepdims=True)
            s1 = jnp.sum(y * y, axis=0, keepdims=True)
            st = jnp.concatenate([s0, s1], axis=0)
            i = pl.program_id(0)

            @pl.when(i == 0)
            def _():
                so_ref[...] = st

            @pl.when(i > 0)
            def _():
                so_ref[...] = so_ref[...] + st

    in_specs = [pl.BlockSpec((mblk, x.shape[1]), lambda i: (i, 0)) for x in xs]
    in_specs += [pl.BlockSpec(w.shape, lambda i: (0, 0)) for w in Ws]
    in_specs.append(pl.BlockSpec((1, Cout), lambda i: (0, 0)))
    args = list(xs) + list(Ws) + [bias.reshape(1, Cout)]
    if has_act:
        C0 = xs[0].shape[1]
        in_specs += [pl.BlockSpec((2, C0), lambda i: (0, 0)),
                     pl.BlockSpec((1, C0), lambda i: (0, 0)),
                     pl.BlockSpec((1, C0), lambda i: (0, 0))]
        args += [act[0], act[1].reshape(1, C0), act[2].reshape(1, C0)]
    out_shape = [jax.ShapeDtypeStruct((M, Cout), jnp.float32)]
    out_specs = [pl.BlockSpec((mblk, Cout), lambda i: (i, 0))]
    if emit_stats:
        out_shape.append(jax.ShapeDtypeStruct((2, Cout), jnp.float32))
        out_specs.append(pl.BlockSpec((2, Cout), lambda i: (0, 0)))
    res = pl.pallas_call(
        body, grid=grid, in_specs=in_specs, out_specs=out_specs,
        out_shape=out_shape, interpret=_INTERPRET)(*args)
    return (res[0], res[1]) if emit_stats else res[0]


def _pool(x, pblk):
    Mp, k, C = x.shape

    def body(x_ref, o_ref):
        o_ref[...] = jnp.max(x_ref[...], axis=1)

    return pl.pallas_call(
        body, grid=(Mp // pblk,),
        in_specs=[pl.BlockSpec((pblk, k, C), lambda i: (i, 0, 0))],
        out_specs=pl.BlockSpec((pblk, C), lambda i: (i, 0)),
        out_shape=jax.ShapeDtypeStruct((Mp, C), jnp.float32),
        interpret=_INTERPRET)(x)


def _sa_group(xyz, cent, feats, fstats, fg, fb, fcount, sblk, k=32):
    B, N, _ = xyz.shape
    S = cent.shape[1]
    has_feats = feats is not None
    xyzT = jnp.transpose(xyz, (0, 2, 1))

    def body(*refs):
        if has_feats:
            (xt_ref, c_ref, f_ref, st_ref, g_ref, b_ref,
             rel_ref, gf_ref) = refs
        else:
            xt_ref, c_ref, rel_ref = refs
        cent_t = c_ref[0]
        cx = cent_t[:, 0:1]
        cy = cent_t[:, 1:2]
        cz = cent_t[:, 2:3]
        px = xt_ref[0][0:1, :]
        py = xt_ref[0][1:2, :]
        pz = xt_ref[0][2:3, :]
        dx = cx - px
        dy = cy - py
        dz = cz - pz
        d = dx * dx + dy * dy + dz * dz
        iota = lax.broadcasted_iota(jnp.int32, (sblk, N), 1)
        if has_feats:
            fact = _bn_relu(f_ref[0], st_ref[...], g_ref[...], b_ref[...],
                            fcount)
        inf = jnp.float32(jnp.inf)
        for j in range(k):
            minv = jnp.min(d, axis=1, keepdims=True)
            ismin = d == minv
            idx = jnp.min(jnp.where(ismin, iota, N), axis=1, keepdims=True)
            sel = iota == idx
            gx = jnp.min(jnp.where(sel, px, inf), axis=1, keepdims=True)
            gy = jnp.min(jnp.where(sel, py, inf), axis=1, keepdims=True)
            gz = jnp.min(jnp.where(sel, pz, inf), axis=1, keepdims=True)
            rel_ref[0, :, j, :] = (
                jnp.concatenate([gx, gy, gz], axis=1) - cent_t)
            if has_feats:
                oh = sel.astype(jnp.float32)
                gf_ref[0, :, j, :] = jnp.dot(
                    oh, fact, preferred_element_type=jnp.float32,
                    precision=lax.Precision.HIGHEST)
            d = jnp.where(sel, inf, d)

    in_specs = [pl.BlockSpec((1, 3, N), lambda b, s: (b, 0, 0)),
                pl.BlockSpec((1, sblk, 3), lambda b, s: (b, s, 0))]
    args = [xyzT, cent]
    out_shape = [jax.ShapeDtypeStruct((B, S, k, 3), jnp.float32)]
    out_specs = [pl.BlockSpec((1, sblk, k, 3), lambda b, s: (b, s, 0, 0))]
    if has_feats:
        Cf = feats.shape[2]
        in_specs += [pl.BlockSpec((1, N, Cf), lambda b, s: (b, 0, 0)),
                     pl.BlockSpec((2, Cf), lambda b, s: (0, 0)),
                     pl.BlockSpec((1, Cf), lambda b, s: (0, 0)),
                     pl.BlockSpec((1, Cf), lambda b, s: (0, 0))]
        args += [feats, fstats, fg.reshape(1, Cf), fb.reshape(1, Cf)]
        out_shape.append(jax.ShapeDtypeStruct((B, S, k, Cf), jnp.float32))
        out_specs.append(pl.BlockSpec((1, sblk, k, Cf),
                                      lambda b, s: (b, s, 0, 0)))
    res = pl.pallas_call(
        body, grid=(B, S // sblk), in_specs=in_specs, out_specs=out_specs,
        out_shape=out_shape, interpret=_INTERPRET)(*args)
    return res if has_feats else (res[0], None)


def _fp(xyz1, xyz2, f2, f2stats, f2g, f2b, f2count,
        f1, f1stats, f1g, f1b, f1count, sblk):
    B, S1, _ = xyz1.shape
    S2 = xyz2.shape[1]
    C2 = f2.shape[2]
    has_f1 = f1 is not None
    xyz2T = jnp.transpose(xyz2, (0, 2, 1))

    def body(*refs):
        if has_f1:
            (x1_ref, x2t_ref, f2_ref, st2_ref, g2_ref, b2_ref,
             f1_ref, st1_ref, g1_ref, b1_ref, oi_ref, of_ref) = refs
        else:
            (x1_ref, x2t_ref, f2_ref, st2_ref, g2_ref, b2_ref,
             oi_ref) = refs
        c1 = x1_ref[0]
        cx = c1[:, 0:1]
        cy = c1[:, 1:2]
        cz = c1[:, 2:3]
        px = x2t_ref[0][0:1, :]
        py = x2t_ref[0][1:2, :]
        pz = x2t_ref[0][2:3, :]
        dx = cx - px
        dy = cy - py
        dz = cz - pz
        d = dx * dx + dy * dy + dz * dz
        iota = lax.broadcasted_iota(jnp.int32, (sblk, S2), 1)
        wm = jnp.zeros((sblk, S2), jnp.float32)
        wsum = jnp.zeros((sblk, 1), jnp.float32)
        for j in range(3):
            minv = jnp.min(d, axis=1, keepdims=True)
            ismin = d == minv
            idx = jnp.min(jnp.where(ismin, iota, S2), axis=1, keepdims=True)
            w = 1.0 / (jnp.maximum(minv, 1e-10) + 1e-8)
            wm = wm + w * (iota == idx).astype(jnp.float32)
            wsum = wsum + w
            d = jnp.where(iota == idx, jnp.inf, d)
        f2act = _bn_relu(f2_ref[0], st2_ref[...], g2_ref[...], b2_ref[...],
                         f2count)
        interp = jnp.dot(wm, f2act, preferred_element_type=jnp.float32, precision=lax.Precision.HIGHEST)
        oi_ref[0] = interp / wsum
        if has_f1:
            of_ref[0] = _bn_relu(f1_ref[0], st1_ref[...], g1_ref[...],
                                 b1_ref[...], f1count)

    in_specs = [pl.BlockSpec((1, sblk, 3), lambda b, s: (b, s, 0)),
                pl.BlockSpec((1, 3, S2), lambda b, s: (b, 0, 0)),
                pl.BlockSpec((1, S2, C2), lambda b, s: (b, 0, 0)),
                pl.BlockSpec((2, C2), lambda b, s: (0, 0)),
                pl.BlockSpec((1, C2), lambda b, s: (0, 0)),
                pl.BlockSpec((1, C2), lambda b, s: (0, 0))]
    args = [xyz1, xyz2T, f2, f2stats, f2g.reshape(1, C2), f2b.reshape(1, C2)]
    out_shape = [jax.ShapeDtypeStruct((B, S1, C2), jnp.float32)]
    out_specs = [pl.BlockSpec((1, sblk, C2), lambda b, s: (b, s, 0))]
    if has_f1:
        C1 = f1.shape[2]
        in_specs += [pl.BlockSpec((1, sblk, C1), lambda b, s: (b, s, 0)),
                     pl.BlockSpec((2, C1), lambda b, s: (0, 0)),
                     pl.BlockSpec((1, C1), lambda b, s: (0, 0)),
                     pl.BlockSpec((1, C1), lambda b, s: (0, 0))]
        args += [f1, f1stats, f1g.reshape(1, C1), f1b.reshape(1, C1)]
        out_shape.append(jax.ShapeDtypeStruct((B, S1, C1), jnp.float32))
        out_specs.append(pl.BlockSpec((1, sblk, C1), lambda b, s: (b, s, 0)))
    res = pl.pallas_call(
        body, grid=(B, S1 // sblk), in_specs=in_specs, out_specs=out_specs,
        out_shape=out_shape, interpret=_INTERPRET)(*args)
    return res if has_f1 else (res[0], None)


def kernel(pts, params):
    B, N, _ = pts.shape
    xyz = pts[:, :, :3]
    np1, np2, k = 1024, 256, 32
    ncls = params['head_w2'].shape[1]

    x1 = xyz[:, :: N // np1, :][:, :np1, :]
    rel1, _ = _sa_group(xyz, x1, None, None, None, None, None, sblk=64, k=k)
    m1 = B * np1 * k
    W, bb, g, bt = params['sa1'][0]
    y, st = _mlp([rel1.reshape(m1, 3)], [W], bb, None, True, 2048)
    act = (st, g, bt, float(m1))
    W, bb, g, bt = params['sa1'][1]
    y, st = _mlp([y], [W], bb, act, True, 2048)
    act = (st, g, bt, float(m1))
    W, bb, g13, bt13 = params['sa1'][2]
    y, st13 = _mlp([y], [W], bb, act, True, 2048)
    c1out = W.shape[1]
    f1raw = _pool(y.reshape(B * np1, k, c1out), 128)

    x2 = x1[:, :: np1 // np2, :][:, :np2, :]
    rel2, gf2 = _sa_group(x1, x2, f1raw.reshape(B, np1, c1out),
                          st13, g13, bt13, float(m1), sblk=64, k=k)
    m2 = B * np2 * k
    W, bb, g, bt = params['sa2'][0]
    y, st = _mlp([rel2.reshape(m2, 3), gf2.reshape(m2, c1out)],
                 [W[:3], W[3:]], bb, None, True, 2048)
    act = (st, g, bt, float(m2))
    W, bb, g, bt = params['sa2'][1]
    y, st = _mlp([y], [W], bb, act, True, 2048)
    act = (st, g, bt, float(m2))
    W, bb, g23, bt23 = params['sa2'][2]
    y, st23 = _mlp([y], [W], bb, act, True, 2048)
    c2out = W.shape[1]
    f2raw = _pool(y.reshape(B * np2, k, c2out), 128)

    interp1, f1a = _fp(x1, x2, f2raw.reshape(B, np2, c2out),
                       st23, g23, bt23, float(m2),
                       f1raw.reshape(B, np1, c1out),
                       st13, g13, bt13, float(m1), sblk=256)
    mf1 = B * np1
    W, bb, g, bt = params['fp1'][0]
    y, st = _mlp([f1a.reshape(mf1, c1out), interp1.reshape(mf1, c2out)],
                 [W[:c1out], W[c1out:]], bb, None, True, 1024)
    act = (st, g, bt, float(mf1))
    W, bb, gf12, btf12 = params['fp1'][1]
    z, stf12 = _mlp([y], [W], bb, act, True, 1024)
    cfp1 = W.shape[1]

    interp2, _ = _fp(xyz, x1, z.reshape(B, np1, cfp1),
                     stf12, gf12, btf12, float(mf1),
                     None, None, None, None, None, sblk=512)
    mf2 = B * N
    W, bb, g, bt = params['fp2'][0]
    y, st = _mlp([interp2.reshape(mf2, cfp1)], [W], bb, None, True, 2048)
    act = (st, g, bt, float(mf2))
    W, bb, gf22, btf22 = params['fp2'][1]
    y, st = _mlp([y], [W], bb, act, True, 2048)

    act = (st, gf22, btf22, float(mf2))
    h, sth = _mlp([y], [params['head_w1']], params['head_b1'], act, True, 2048)
    act = (sth, params['head_g'], params['head_bt'], float(mf2))
    o = _mlp([h], [params['head_w2']], params['head_b2'], act, False, 2048)
    return o.reshape(B, N, ncls)
